# baseline (speedup 1.0000x reference)
import functools

import jax
import jax.numpy as jnp
from jax.experimental import pallas as pl
from jax.experimental.pallas import tpu as pltpu
from jax.experimental.pallas import tpu_sc as plsc

_B, _H, _S = 32, 1024, 576
_ND, _ED, _K = 4, 256, 1024
_COMMIT = 0.25
_BS = _B * _S
_TB = 512
_NSTEPS = _BS // _TB
_NELEMS = _B * _H * _S
_NW = 32
_EPW = _ED // 8
_L = 16

_PREC_DIST = jax.lax.Precision.DEFAULT


def _argmin_body(zrow_ref, emb_ref, idx_ref, e2_ref):
    i = pl.program_id(0)

    @pl.when(i == 0)
    def _init():
        e2_ref[...] = jnp.sum(emb_ref[...] * emb_ref[...], axis=1)

    for d in range(_ND):
        zd = zrow_ref[:, d * _ED:(d + 1) * _ED]
        emb = emb_ref[d]
        e2 = e2_ref[d]
        z2 = jnp.sum(zd * zd, axis=1)
        g = jax.lax.dot_general(
            zd, emb, (((1,), (0,)), ((), ())),
            preferred_element_type=jnp.float32,
            precision=_PREC_DIST)
        scores = (z2[:, None] + e2[None, :]) - 2.0 * g
        m = jnp.min(scores, axis=1, keepdims=True)
        kiota = jax.lax.broadcasted_iota(jnp.int32, (_TB, _K), 1)
        idx_ref[d] = jnp.min(
            jnp.where(scores == m, kiota, _K), axis=1)


def _tc_argmin(zrow, embeddings):
    return pl.pallas_call(
        _argmin_body,
        grid=(_NSTEPS,),
        in_specs=[
            pl.BlockSpec((_TB, _ND * _ED), lambda i: (i, 0)),
            pl.BlockSpec((_ND, _ED, _K), lambda i: (0, 0, 0)),
        ],
        out_specs=pl.BlockSpec((8, _TB), lambda i: (0, i)),
        out_shape=jax.ShapeDtypeStruct((8, _BS), jnp.int32),
        scratch_shapes=[pltpu.VMEM((_ND, _K), jnp.float32)],
    )(zrow, embeddings)


def _gather_body(emb_ref, idx_ref, z_ref, out_ref, part_ref,
                 idx_v, tab_v, z_v, o_v, acc_v):
    wid = jax.lax.axis_index("s") * 2 + jax.lax.axis_index("c")
    d = wid // 8
    e0 = (wid % 8) * _EPW
    pltpu.sync_copy(idx_ref.at[d], idx_v)
    pltpu.sync_copy(emb_ref.at[pl.ds(d * _ED + e0, _EPW)], tab_v)

    acc_v[...] = jnp.zeros((_L,), jnp.float32)

    def j_body(j, carry):
        pltpu.sync_copy(z_ref.at[wid, pl.ds(j * _EPW, _EPW)], z_v)
        jv = jnp.full((_L,), j, jnp.int32)

        def r_body(r, carry):
            acc = acc_v[...]
            for s16 in range(_S // _L):
                t0 = r * _S + s16 * _L
                i16 = idx_v[pl.ds(t0, _L)]
                z16 = z_v[r, pl.ds(s16 * _L, _L)]
                g = plsc.load_gather(tab_v, [jv, i16])
                dfr = z16 - g
                acc = acc + dfr * dfr
                o_v[r, pl.ds(s16 * _L, _L)] = g
            acc_v[...] = acc
            return carry

        carry = jax.lax.fori_loop(0, _EPW, r_body, carry)
        pltpu.sync_copy(o_v, out_ref.at[wid, pl.ds(j * _EPW, _EPW)])
        return carry

    jax.lax.fori_loop(0, _EPW, j_body, jnp.int32(0))
    pltpu.sync_copy(acc_v, part_ref.at[wid])


_sc_gather = functools.partial(
    pl.kernel,
    out_type=[jax.ShapeDtypeStruct((_B, _H, _S), jnp.float32),
              jax.ShapeDtypeStruct((_NW, _L), jnp.float32)],
    mesh=plsc.VectorSubcoreMesh(core_axis_name="c", subcore_axis_name="s"),
    scratch_types=[
        pltpu.VMEM((_BS,), jnp.int32),
        pltpu.VMEM((_EPW, _K), jnp.float32),
        pltpu.VMEM((_EPW, _S), jnp.float32),
        pltpu.VMEM((_EPW, _S), jnp.float32),
        pltpu.VMEM((_L,), jnp.float32),
    ],
    compiler_params=pltpu.CompilerParams(needs_layout_passes=False),
)(_gather_body)


def kernel(inputs, embeddings):
    zrow = inputs.reshape(_BS, _ND * _ED)
    idx = _tc_argmin(zrow, embeddings)
    emb_flat = embeddings.reshape(_ND * _ED, _K)
    output, parts = _sc_gather(emb_flat, idx, inputs)
    commit_loss = jnp.sum(parts) / jnp.float32(_NELEMS)
    return (output, _COMMIT * commit_loss, commit_loss, jnp.array(0))


# speedup vs baseline: 1.0019x; 1.0019x over previous
import functools

import jax
import jax.numpy as jnp
from jax.experimental import pallas as pl
from jax.experimental.pallas import tpu as pltpu
from jax.experimental.pallas import tpu_sc as plsc

_B, _H, _S = 32, 1024, 576
_ND, _ED, _K = 4, 256, 1024
_COMMIT = 0.25
_BS = _B * _S
_TB = 512
_NSTEPS = _BS // _TB
_NELEMS = _B * _H * _S
_NW = 32
_EPW = _ED // 8
_L = 16

_PREC_DIST = jax.lax.Precision.DEFAULT


def _argmin_body(zrow_ref, emb_ref, idx_ref, e2_ref):
    i = pl.program_id(0)

    @pl.when(i == 0)
    def _init():
        e2_ref[...] = jnp.sum(emb_ref[...] * emb_ref[...], axis=1)

    for d in range(_ND):
        zd = zrow_ref[:, d * _ED:(d + 1) * _ED]
        emb = emb_ref[d]
        e2 = e2_ref[d]
        z2 = jnp.sum(zd * zd, axis=1)
        g = jax.lax.dot_general(
            zd, emb, (((1,), (0,)), ((), ())),
            preferred_element_type=jnp.float32,
            precision=_PREC_DIST)
        scores = (z2[:, None] + e2[None, :]) - 2.0 * g
        m = jnp.min(scores, axis=1, keepdims=True)
        kiota = jax.lax.broadcasted_iota(jnp.int32, (_TB, _K), 1)
        idx_ref[d] = jnp.min(
            jnp.where(scores == m, kiota, _K), axis=1)


def _tc_argmin(zrow, embeddings):
    return pl.pallas_call(
        _argmin_body,
        grid=(_NSTEPS,),
        in_specs=[
            pl.BlockSpec((_TB, _ND * _ED), lambda i: (i, 0)),
            pl.BlockSpec((_ND, _ED, _K), lambda i: (0, 0, 0)),
        ],
        out_specs=pl.BlockSpec((8, _TB), lambda i: (0, i)),
        out_shape=jax.ShapeDtypeStruct((8, _BS), jnp.int32),
        scratch_shapes=[pltpu.VMEM((_ND, _K), jnp.float32)],
    )(zrow, embeddings)


_NACC = 6


def _gather_body(emb_ref, idx_ref, z_ref, out_ref, part_ref,
                 idx_v, tab_v, z_v, o_v, acc_v):
    wid = jax.lax.axis_index("s") * 2 + jax.lax.axis_index("c")
    d = wid // 8
    e0 = (wid % 8) * _EPW
    pltpu.sync_copy(idx_ref.at[d], idx_v)
    pltpu.sync_copy(emb_ref.at[pl.ds(d * _ED + e0, _EPW)], tab_v)

    for a in range(_NACC):
        acc_v[a] = jnp.zeros((_L,), jnp.float32)

    def j_body(j, carry):
        pltpu.sync_copy(z_ref.at[wid, pl.ds(j * _EPW, _EPW)], z_v)
        jv = jnp.full((_L,), j, jnp.int32)

        def r_body(r, carry):
            accs = [acc_v[a] for a in range(_NACC)]
            for s16 in range(_S // _L):
                t0 = r * _S + s16 * _L
                i16 = idx_v[pl.ds(t0, _L)]
                z16 = z_v[r, pl.ds(s16 * _L, _L)]
                g = plsc.load_gather(tab_v, [jv, i16])
                dfr = z16 - g
                accs[s16 % _NACC] = accs[s16 % _NACC] + dfr * dfr
                o_v[r, pl.ds(s16 * _L, _L)] = g
            for a in range(_NACC):
                acc_v[a] = accs[a]
            return carry

        carry = jax.lax.fori_loop(0, _EPW, r_body, carry)
        pltpu.sync_copy(o_v, out_ref.at[wid, pl.ds(j * _EPW, _EPW)])
        return carry

    jax.lax.fori_loop(0, _EPW, j_body, jnp.int32(0))
    acc_v[0] = ((acc_v[0] + acc_v[1]) + (acc_v[2] + acc_v[3])
                + (acc_v[4] + acc_v[5]))
    pltpu.sync_copy(acc_v.at[0], part_ref.at[wid])


_sc_gather = functools.partial(
    pl.kernel,
    out_type=[jax.ShapeDtypeStruct((_B, _H, _S), jnp.float32),
              jax.ShapeDtypeStruct((_NW, _L), jnp.float32)],
    mesh=plsc.VectorSubcoreMesh(core_axis_name="c", subcore_axis_name="s"),
    scratch_types=[
        pltpu.VMEM((_BS,), jnp.int32),
        pltpu.VMEM((_EPW, _K), jnp.float32),
        pltpu.VMEM((_EPW, _S), jnp.float32),
        pltpu.VMEM((_EPW, _S), jnp.float32),
        pltpu.VMEM((_NACC, _L), jnp.float32),
    ],
    compiler_params=pltpu.CompilerParams(needs_layout_passes=False),
)(_gather_body)


def kernel(inputs, embeddings):
    zrow = inputs.reshape(_BS, _ND * _ED)
    idx = _tc_argmin(zrow, embeddings)
    emb_flat = embeddings.reshape(_ND * _ED, _K)
    output, parts = _sc_gather(emb_flat, idx, inputs)
    commit_loss = jnp.sum(parts) / jnp.float32(_NELEMS)
    return (output, _COMMIT * commit_loss, commit_loss, jnp.array(0))


# speedup vs baseline: 1.1090x; 1.1069x over previous
import functools

import jax
import jax.numpy as jnp
from jax.experimental import pallas as pl
from jax.experimental.pallas import tpu as pltpu
from jax.experimental.pallas import tpu_sc as plsc

_B, _H, _S = 32, 1024, 576
_ND, _ED, _K = 4, 256, 1024
_COMMIT = 0.25
_BS = _B * _S
_TB = 512
_NSTEPS = _BS // _TB
_NELEMS = _B * _H * _S
_NW = 32
_EPW = _ED // 8
_L = 16

_PREC_DIST = jax.lax.Precision.DEFAULT


def _argmin_body(zrow_ref, emb_ref, idx_ref, e2_ref):
    i = pl.program_id(0)

    @pl.when(i == 0)
    def _init():
        e2_ref[...] = jnp.sum(emb_ref[...] * emb_ref[...], axis=1)

    for d in range(_ND):
        zd = zrow_ref[:, d * _ED:(d + 1) * _ED]
        emb = emb_ref[d]
        e2 = e2_ref[d]
        z2 = jnp.sum(zd * zd, axis=1)
        g = jax.lax.dot_general(
            zd, emb, (((1,), (0,)), ((), ())),
            preferred_element_type=jnp.float32,
            precision=_PREC_DIST)
        scores = (z2[:, None] + e2[None, :]) - 2.0 * g
        m = jnp.min(scores, axis=1, keepdims=True)
        kiota = jax.lax.broadcasted_iota(jnp.int32, (_TB, _K), 1)
        idx_ref[d] = jnp.min(
            jnp.where(scores == m, kiota, _K), axis=1)


def _tc_argmin(zrow, embeddings):
    return pl.pallas_call(
        _argmin_body,
        grid=(_NSTEPS,),
        in_specs=[
            pl.BlockSpec((_TB, _ND * _ED), lambda i: (i, 0)),
            pl.BlockSpec((_ND, _ED, _K), lambda i: (0, 0, 0)),
        ],
        out_specs=pl.BlockSpec((8, _TB), lambda i: (0, i)),
        out_shape=jax.ShapeDtypeStruct((8, _BS), jnp.int32),
        scratch_shapes=[pltpu.VMEM((_ND, _K), jnp.float32)],
    )(zrow, embeddings)


_NACC = 6


def _gather_body(emb_ref, idx_ref, z_ref, out_ref, part_ref,
                 idx_v, tab_v, z_v, o_v, acc_v):
    wid = jax.lax.axis_index("s") * 2 + jax.lax.axis_index("c")
    d = wid // 8
    e0 = (wid % 8) * _EPW
    pltpu.sync_copy(idx_ref.at[d], idx_v)
    pltpu.sync_copy(emb_ref.at[pl.ds(d * _ED + e0, _EPW)], tab_v)

    for a in range(_NACC):
        acc_v[a] = jnp.zeros((_L,), jnp.float32)

    def j_body(j, carry):
        pltpu.sync_copy(z_ref.at[wid, pl.ds(j * _EPW, _EPW)], z_v)
        jv = jnp.full((_L,), j, jnp.int32)

        def r_body(r, accs):
            accs = list(accs)
            for s16 in range(_S // _L):
                t0 = r * _S + s16 * _L
                i16 = idx_v[pl.ds(t0, _L)]
                z16 = z_v[r, pl.ds(s16 * _L, _L)]
                g = plsc.load_gather(tab_v, [jv, i16])
                dfr = z16 - g
                accs[s16 % _NACC] = accs[s16 % _NACC] + dfr * dfr
                o_v[r, pl.ds(s16 * _L, _L)] = g
            return tuple(accs)

        accs = plsc.parallel_loop(
            0, _EPW, unroll=2,
            carry=tuple(acc_v[a] for a in range(_NACC)))(r_body)
        for a in range(_NACC):
            acc_v[a] = accs[a]
        pltpu.sync_copy(o_v, out_ref.at[wid, pl.ds(j * _EPW, _EPW)])
        return carry

    jax.lax.fori_loop(0, _EPW, j_body, jnp.int32(0))
    acc_v[0] = ((acc_v[0] + acc_v[1]) + (acc_v[2] + acc_v[3])
                + (acc_v[4] + acc_v[5]))
    pltpu.sync_copy(acc_v.at[0], part_ref.at[wid])


_sc_gather = functools.partial(
    pl.kernel,
    out_type=[jax.ShapeDtypeStruct((_B, _H, _S), jnp.float32),
              jax.ShapeDtypeStruct((_NW, _L), jnp.float32)],
    mesh=plsc.VectorSubcoreMesh(core_axis_name="c", subcore_axis_name="s"),
    scratch_types=[
        pltpu.VMEM((_BS,), jnp.int32),
        pltpu.VMEM((_EPW, _K), jnp.float32),
        pltpu.VMEM((_EPW, _S), jnp.float32),
        pltpu.VMEM((_EPW, _S), jnp.float32),
        pltpu.VMEM((_NACC, _L), jnp.float32),
    ],
    compiler_params=pltpu.CompilerParams(needs_layout_passes=False),
)(_gather_body)


def kernel(inputs, embeddings):
    zrow = inputs.reshape(_BS, _ND * _ED)
    idx = _tc_argmin(zrow, embeddings)
    emb_flat = embeddings.reshape(_ND * _ED, _K)
    output, parts = _sc_gather(emb_flat, idx, inputs)
    commit_loss = jnp.sum(parts) / jnp.float32(_NELEMS)
    return (output, _COMMIT * commit_loss, commit_loss, jnp.array(0))


# speedup vs baseline: 1.5575x; 1.4043x over previous
import functools

import jax
import jax.numpy as jnp
from jax.experimental import pallas as pl
from jax.experimental.pallas import tpu as pltpu
from jax.experimental.pallas import tpu_sc as plsc

_B, _H, _S = 32, 1024, 576
_ND, _ED, _K = 4, 256, 1024
_COMMIT = 0.25
_BS = _B * _S
_TB = 512
_NSTEPS = _BS // _TB
_NELEMS = _B * _H * _S
_NW = 32
_EPW = _ED // 8
_L = 16

_PREC_DIST = jax.lax.Precision.DEFAULT


def _argmin_body(zrow_ref, emb_ref, idx_ref, e2_ref):
    i = pl.program_id(0)

    @pl.when(i == 0)
    def _init():
        e2_ref[...] = jnp.sum(emb_ref[...] * emb_ref[...], axis=1)

    for d in range(_ND):
        zd = zrow_ref[:, d * _ED:(d + 1) * _ED]
        emb = emb_ref[d]
        e2 = e2_ref[d]
        z2 = jnp.sum(zd * zd, axis=1)
        g = jax.lax.dot_general(
            zd, emb, (((1,), (0,)), ((), ())),
            preferred_element_type=jnp.float32,
            precision=_PREC_DIST)
        scores = (z2[:, None] + e2[None, :]) - 2.0 * g
        m = jnp.min(scores, axis=1, keepdims=True)
        kiota = jax.lax.broadcasted_iota(jnp.int32, (_TB, _K), 1)
        idx_ref[d] = jnp.min(
            jnp.where(scores == m, kiota, _K), axis=1)


def _tc_argmin(zrow, embeddings):
    return pl.pallas_call(
        _argmin_body,
        grid=(_NSTEPS,),
        in_specs=[
            pl.BlockSpec((_TB, _ND * _ED), lambda i: (i, 0)),
            pl.BlockSpec((_ND, _ED, _K), lambda i: (0, 0, 0)),
        ],
        out_specs=pl.BlockSpec((8, _TB), lambda i: (0, i)),
        out_shape=jax.ShapeDtypeStruct((8, _BS), jnp.int32),
        scratch_shapes=[pltpu.VMEM((_ND, _K), jnp.float32)],
    )(zrow, embeddings)


_NACC = 6


def _gather_body(emb_ref, idx_ref, out_ref, idx_v, tab_v, o_v):
    wid = jax.lax.axis_index("s") * 2 + jax.lax.axis_index("c")
    d = wid // 8
    e0 = (wid % 8) * _EPW
    pltpu.sync_copy(idx_ref.at[d], idx_v)
    pltpu.sync_copy(emb_ref.at[pl.ds(d * _ED + e0, _EPW)], tab_v)

    def j_body(j, carry):
        jv = jnp.full((_L,), j, jnp.int32)

        def r_body(r, inner):
            for s16 in range(_S // _L):
                t0 = r * _S + s16 * _L
                i16 = idx_v[pl.ds(t0, _L)]
                g = plsc.load_gather(tab_v, [jv, i16])
                o_v[r, pl.ds(s16 * _L, _L)] = g
            return inner

        plsc.parallel_loop(0, _EPW, unroll=2, carry=jnp.int32(0))(r_body)
        pltpu.sync_copy(o_v, out_ref.at[wid, pl.ds(j * _EPW, _EPW)])
        return carry

    jax.lax.fori_loop(0, _EPW, j_body, jnp.int32(0))


_sc_gather = functools.partial(
    pl.kernel,
    out_type=jax.ShapeDtypeStruct((_B, _H, _S), jnp.float32),
    mesh=plsc.VectorSubcoreMesh(core_axis_name="c", subcore_axis_name="s"),
    scratch_types=[
        pltpu.VMEM((_BS,), jnp.int32),
        pltpu.VMEM((_EPW, _K), jnp.float32),
        pltpu.VMEM((_EPW, _S), jnp.float32),
    ],
    compiler_params=pltpu.CompilerParams(needs_layout_passes=False),
)(_gather_body)


def _commit_body(z_ref, enc_ref, commit_ref, loss_ref, acc_ref):
    i = pl.program_id(0)

    @pl.when(i == 0)
    def _init():
        acc_ref[0, 0] = 0.0

    diff = z_ref[...] - enc_ref[...]
    acc_ref[0, 0] += jnp.sum(diff * diff)

    @pl.when(i == _B - 1)
    def _fin():
        c = acc_ref[0, 0] / jnp.float32(_NELEMS)
        commit_ref[0, 0] = c
        loss_ref[0, 0] = jnp.float32(_COMMIT) * c


def _tc_commit(inputs, output):
    return pl.pallas_call(
        _commit_body,
        grid=(_B,),
        in_specs=[
            pl.BlockSpec((1, _H, _S), lambda i: (i, 0, 0)),
            pl.BlockSpec((1, _H, _S), lambda i: (i, 0, 0)),
        ],
        out_specs=[
            pl.BlockSpec(memory_space=pltpu.SMEM),
            pl.BlockSpec(memory_space=pltpu.SMEM),
        ],
        out_shape=[
            jax.ShapeDtypeStruct((1, 1), jnp.float32),
            jax.ShapeDtypeStruct((1, 1), jnp.float32),
        ],
        scratch_shapes=[pltpu.SMEM((1, 1), jnp.float32)],
    )(inputs, output)


def kernel(inputs, embeddings):
    zrow = inputs.reshape(_BS, _ND * _ED)
    idx = _tc_argmin(zrow, embeddings)
    emb_flat = embeddings.reshape(_ND * _ED, _K)
    output = _sc_gather(emb_flat, idx)
    commit, loss = _tc_commit(inputs, output)
    return (output, loss[0, 0], commit[0, 0], jnp.array(0))


# speedup vs baseline: 1.5863x; 1.0185x over previous
import functools

import jax
import jax.numpy as jnp
from jax.experimental import pallas as pl
from jax.experimental.pallas import tpu as pltpu
from jax.experimental.pallas import tpu_sc as plsc

_B, _H, _S = 32, 1024, 576
_ND, _ED, _K = 4, 256, 1024
_COMMIT = 0.25
_BS = _B * _S
_TB = 1024
_NSTEPS = _BS // _TB
_NELEMS = _B * _H * _S
_NW = 32
_EPW = _ED // 8
_L = 16

_PREC_DIST = jax.lax.Precision.DEFAULT


def _argmin_body(zrow_ref, emb_ref, idx_ref, e2_ref):
    i = pl.program_id(0)

    @pl.when(i == 0)
    def _init():
        e2_ref[...] = jnp.sum(emb_ref[...] * emb_ref[...], axis=1)

    for d in range(_ND):
        zd = zrow_ref[:, d * _ED:(d + 1) * _ED]
        emb = emb_ref[d]
        e2 = e2_ref[d]
        z2 = jnp.sum(zd * zd, axis=1)
        g = jax.lax.dot_general(
            zd, emb, (((1,), (0,)), ((), ())),
            preferred_element_type=jnp.float32,
            precision=_PREC_DIST)
        scores = (z2[:, None] + e2[None, :]) - 2.0 * g
        m = jnp.min(scores, axis=1, keepdims=True)
        kiota = jax.lax.broadcasted_iota(jnp.int32, (_TB, _K), 1)
        idx_ref[d] = jnp.min(
            jnp.where(scores == m, kiota, _K), axis=1)


def _tc_argmin(zrow, embeddings):
    return pl.pallas_call(
        _argmin_body,
        grid=(_NSTEPS,),
        in_specs=[
            pl.BlockSpec((_TB, _ND * _ED), lambda i: (i, 0)),
            pl.BlockSpec((_ND, _ED, _K), lambda i: (0, 0, 0)),
        ],
        out_specs=pl.BlockSpec((8, _TB), lambda i: (0, i)),
        out_shape=jax.ShapeDtypeStruct((8, _BS), jnp.int32),
        scratch_shapes=[pltpu.VMEM((_ND, _K), jnp.float32)],
    )(zrow, embeddings)


_NACC = 6


def _gather_body(emb_ref, idx_ref, out_ref, idx_v, tab_v, o_v):
    wid = jax.lax.axis_index("s") * 2 + jax.lax.axis_index("c")
    d = wid // 8
    e0 = (wid % 8) * _EPW
    pltpu.sync_copy(idx_ref.at[d], idx_v)
    pltpu.sync_copy(emb_ref.at[pl.ds(d * _ED + e0, _EPW)], tab_v)

    def j_body(j, carry):
        jv = jnp.full((_L,), j, jnp.int32)

        def r_body(r, inner):
            for s16 in range(_S // _L):
                t0 = r * _S + s16 * _L
                i16 = idx_v[pl.ds(t0, _L)]
                g = plsc.load_gather(tab_v, [jv, i16])
                o_v[r, pl.ds(s16 * _L, _L)] = g
            return inner

        plsc.parallel_loop(0, _EPW, unroll=4, carry=jnp.int32(0))(r_body)
        pltpu.sync_copy(o_v, out_ref.at[wid, pl.ds(j * _EPW, _EPW)])
        return carry

    jax.lax.fori_loop(0, _EPW, j_body, jnp.int32(0))


_sc_gather = functools.partial(
    pl.kernel,
    out_type=jax.ShapeDtypeStruct((_B, _H, _S), jnp.float32),
    mesh=plsc.VectorSubcoreMesh(core_axis_name="c", subcore_axis_name="s"),
    scratch_types=[
        pltpu.VMEM((_BS,), jnp.int32),
        pltpu.VMEM((_EPW, _K), jnp.float32),
        pltpu.VMEM((_EPW, _S), jnp.float32),
    ],
    compiler_params=pltpu.CompilerParams(needs_layout_passes=False),
)(_gather_body)


def _commit_body(z_ref, enc_ref, commit_ref, loss_ref, acc_ref):
    i = pl.program_id(0)

    @pl.when(i == 0)
    def _init():
        acc_ref[0, 0] = 0.0

    diff = z_ref[...] - enc_ref[...]
    acc_ref[0, 0] += jnp.sum(diff * diff)

    @pl.when(i == _B - 1)
    def _fin():
        c = acc_ref[0, 0] / jnp.float32(_NELEMS)
        commit_ref[0, 0] = c
        loss_ref[0, 0] = jnp.float32(_COMMIT) * c


def _tc_commit(inputs, output):
    return pl.pallas_call(
        _commit_body,
        grid=(_B,),
        in_specs=[
            pl.BlockSpec((1, _H, _S), lambda i: (i, 0, 0)),
            pl.BlockSpec((1, _H, _S), lambda i: (i, 0, 0)),
        ],
        out_specs=[
            pl.BlockSpec(memory_space=pltpu.SMEM),
            pl.BlockSpec(memory_space=pltpu.SMEM),
        ],
        out_shape=[
            jax.ShapeDtypeStruct((1, 1), jnp.float32),
            jax.ShapeDtypeStruct((1, 1), jnp.float32),
        ],
        scratch_shapes=[pltpu.SMEM((1, 1), jnp.float32)],
    )(inputs, output)


def kernel(inputs, embeddings):
    zrow = inputs.reshape(_BS, _ND * _ED)
    idx = _tc_argmin(zrow, embeddings)
    emb_flat = embeddings.reshape(_ND * _ED, _K)
    output = _sc_gather(emb_flat, idx)
    commit, loss = _tc_commit(inputs, output)
    return (output, loss[0, 0], commit[0, 0], jnp.array(0))


# speedup vs baseline: 1.6400x; 1.0338x over previous
import functools

import jax
import jax.numpy as jnp
from jax.experimental import pallas as pl
from jax.experimental.pallas import tpu as pltpu
from jax.experimental.pallas import tpu_sc as plsc

_B, _H, _S = 32, 1024, 576
_ND, _ED, _K = 4, 256, 1024
_COMMIT = 0.25
_BS = _B * _S
_TB = 1024
_NSTEPS = _BS // _TB
_NELEMS = _B * _H * _S
_NW = 32
_EPW = _ED // 8
_L = 16

_PREC_DIST = jax.lax.Precision.DEFAULT


def _argmin_body(zrow_ref, emb_ref, idx_ref, e2_ref):
    i = pl.program_id(0)

    @pl.when(i == 0)
    def _init():
        e2_ref[...] = jnp.sum(emb_ref[...] * emb_ref[...], axis=1)

    for d in range(_ND):
        zd = zrow_ref[:, d * _ED:(d + 1) * _ED]
        emb = emb_ref[d]
        e2 = e2_ref[d]
        z2 = jnp.sum(zd * zd, axis=1)
        g = jax.lax.dot_general(
            zd, emb, (((1,), (0,)), ((), ())),
            preferred_element_type=jnp.float32,
            precision=_PREC_DIST)
        scores = (z2[:, None] + e2[None, :]) - 2.0 * g
        m = jnp.min(scores, axis=1, keepdims=True)
        kiota = jax.lax.broadcasted_iota(jnp.int32, (_TB, _K), 1)
        idx_ref[d] = jnp.min(
            jnp.where(scores == m, kiota, _K), axis=1)


def _tc_argmin(zrow, embeddings):
    return pl.pallas_call(
        _argmin_body,
        grid=(_NSTEPS,),
        in_specs=[
            pl.BlockSpec((_TB, _ND * _ED), lambda i: (i, 0)),
            pl.BlockSpec((_ND, _ED, _K), lambda i: (0, 0, 0)),
        ],
        out_specs=pl.BlockSpec((8, _TB), lambda i: (0, i)),
        out_shape=jax.ShapeDtypeStruct((8, _BS), jnp.int32),
        scratch_shapes=[pltpu.VMEM((_ND, _K), jnp.float32)],
    )(zrow, embeddings)


def _gather_body(emb_ref, idx_ref, out_ref, idx_v, tab_v, o_v, sem):
    wid = jax.lax.axis_index("s") * 2 + jax.lax.axis_index("c")
    d = wid // 8
    e0 = (wid % 8) * _EPW
    pltpu.sync_copy(idx_ref.at[d], idx_v)
    pltpu.sync_copy(emb_ref.at[pl.ds(d * _ED + e0, _EPW)], tab_v)

    def j_body(j, carry):
        jv = jnp.full((_L,), j, jnp.int32)
        buf = jax.lax.rem(j, 2)

        def r_body(r, inner):
            for s16 in range(_S // _L):
                t0 = r * _S + s16 * _L
                i16 = idx_v[pl.ds(t0, _L)]
                g = plsc.load_gather(tab_v, [jv, i16])
                o_v[buf, r, pl.ds(s16 * _L, _L)] = g
            return inner

        plsc.parallel_loop(0, _EPW, unroll=4, carry=jnp.int32(0))(r_body)

        @pl.when(j > 0)
        def _drain():
            pltpu.make_async_copy(
                o_v.at[1 - buf],
                out_ref.at[wid, pl.ds((j - 1) * _EPW, _EPW)], sem).wait()

        pltpu.make_async_copy(
            o_v.at[buf],
            out_ref.at[wid, pl.ds(j * _EPW, _EPW)], sem).start()
        return carry

    jax.lax.fori_loop(0, _EPW, j_body, jnp.int32(0))
    pltpu.make_async_copy(
        o_v.at[1], out_ref.at[wid, pl.ds((_EPW - 1) * _EPW, _EPW)],
        sem).wait()


_sc_gather = functools.partial(
    pl.kernel,
    out_type=jax.ShapeDtypeStruct((_B, _H, _S), jnp.float32),
    mesh=plsc.VectorSubcoreMesh(core_axis_name="c", subcore_axis_name="s"),
    scratch_types=[
        pltpu.VMEM((_BS,), jnp.int32),
        pltpu.VMEM((_EPW, _K), jnp.float32),
        pltpu.VMEM((2, _EPW, _S), jnp.float32),
        pltpu.SemaphoreType.DMA,
    ],
    compiler_params=pltpu.CompilerParams(needs_layout_passes=False),
)(_gather_body)


def _commit_body(z_ref, enc_ref, commit_ref, loss_ref, acc_ref):
    i = pl.program_id(0)

    @pl.when(i == 0)
    def _init():
        acc_ref[0, 0] = 0.0

    diff = z_ref[...] - enc_ref[...]
    acc_ref[0, 0] += jnp.sum(diff * diff)

    @pl.when(i == _B - 1)
    def _fin():
        c = acc_ref[0, 0] / jnp.float32(_NELEMS)
        commit_ref[0, 0] = c
        loss_ref[0, 0] = jnp.float32(_COMMIT) * c


def _tc_commit(inputs, output):
    return pl.pallas_call(
        _commit_body,
        grid=(_B,),
        in_specs=[
            pl.BlockSpec((1, _H, _S), lambda i: (i, 0, 0)),
            pl.BlockSpec((1, _H, _S), lambda i: (i, 0, 0)),
        ],
        out_specs=[
            pl.BlockSpec(memory_space=pltpu.SMEM),
            pl.BlockSpec(memory_space=pltpu.SMEM),
        ],
        out_shape=[
            jax.ShapeDtypeStruct((1, 1), jnp.float32),
            jax.ShapeDtypeStruct((1, 1), jnp.float32),
        ],
        scratch_shapes=[pltpu.SMEM((1, 1), jnp.float32)],
    )(inputs, output)


def kernel(inputs, embeddings):
    zrow = inputs.reshape(_BS, _ND * _ED)
    idx = _tc_argmin(zrow, embeddings)
    emb_flat = embeddings.reshape(_ND * _ED, _K)
    output = _sc_gather(emb_flat, idx)
    commit, loss = _tc_commit(inputs, output)
    return (output, loss[0, 0], commit[0, 0], jnp.array(0))


# speedup vs baseline: 1.6507x; 1.0065x over previous
import functools

import jax
import jax.numpy as jnp
from jax.experimental import pallas as pl
from jax.experimental.pallas import tpu as pltpu
from jax.experimental.pallas import tpu_sc as plsc

_B, _H, _S = 32, 1024, 576
_ND, _ED, _K = 4, 256, 1024
_COMMIT = 0.25
_BS = _B * _S
_TB = 1024
_NSTEPS = _BS // _TB
_NELEMS = _B * _H * _S
_NW = 32
_EPW = _ED // 8
_L = 16

_PREC_DIST = jax.lax.Precision.DEFAULT


def _argmin_body(zrow_ref, emb_ref, idx_ref, e2_ref):
    i = pl.program_id(0)

    @pl.when(i == 0)
    def _init():
        e2_ref[...] = jnp.sum(emb_ref[...] * emb_ref[...], axis=1)

    for d in range(_ND):
        zd = zrow_ref[:, d * _ED:(d + 1) * _ED]
        emb = emb_ref[d]
        e2 = e2_ref[d]
        z2 = jnp.sum(zd * zd, axis=1)
        g = jax.lax.dot_general(
            zd, emb, (((1,), (0,)), ((), ())),
            preferred_element_type=jnp.float32,
            precision=_PREC_DIST)
        scores = (z2[:, None] + e2[None, :]) - 2.0 * g
        m = jnp.min(scores, axis=1, keepdims=True)
        kiota = jax.lax.broadcasted_iota(jnp.int32, (_TB, _K), 1)
        idx_ref[d] = jnp.min(
            jnp.where(scores == m, kiota, _K), axis=1)


def _tc_argmin(zrow, embeddings):
    return pl.pallas_call(
        _argmin_body,
        grid=(_NSTEPS,),
        in_specs=[
            pl.BlockSpec((_TB, _ND * _ED), lambda i: (i, 0)),
            pl.BlockSpec((_ND, _ED, _K), lambda i: (0, 0, 0)),
        ],
        out_specs=pl.BlockSpec((8, _TB), lambda i: (0, i)),
        out_shape=jax.ShapeDtypeStruct((8, _BS), jnp.int32),
        scratch_shapes=[pltpu.VMEM((_ND, _K), jnp.float32)],
    )(zrow, embeddings)


def _gather_body(emb_ref, idx_ref, out_ref, idx_v, tab_v, o_v, sem):
    wid = jax.lax.axis_index("s") * 2 + jax.lax.axis_index("c")
    d = wid // 8
    e0 = (wid % 8) * _EPW
    pltpu.sync_copy(idx_ref.at[d], idx_v)
    pltpu.sync_copy(emb_ref.at[pl.ds(d * _ED + e0, _EPW)], tab_v)

    def j_body(j, carry):
        jv = jnp.full((_L,), j, jnp.int32)
        buf = jax.lax.rem(j, 2)

        def r_body(r, inner):
            for s16 in range(_S // _L):
                t0 = r * _S + s16 * _L
                i16 = idx_v[pl.ds(t0, _L)]
                g = plsc.load_gather(tab_v, [jv, i16])
                o_v[buf, r, pl.ds(s16 * _L, _L)] = g
            return inner

        plsc.parallel_loop(0, _EPW, unroll=8, carry=jnp.int32(0))(r_body)

        @pl.when(j > 0)
        def _drain():
            pltpu.make_async_copy(
                o_v.at[1 - buf],
                out_ref.at[wid, pl.ds((j - 1) * _EPW, _EPW)], sem).wait()

        pltpu.make_async_copy(
            o_v.at[buf],
            out_ref.at[wid, pl.ds(j * _EPW, _EPW)], sem).start()
        return carry

    jax.lax.fori_loop(0, _EPW, j_body, jnp.int32(0))
    pltpu.make_async_copy(
        o_v.at[1], out_ref.at[wid, pl.ds((_EPW - 1) * _EPW, _EPW)],
        sem).wait()


_sc_gather = functools.partial(
    pl.kernel,
    out_type=jax.ShapeDtypeStruct((_B, _H, _S), jnp.float32),
    mesh=plsc.VectorSubcoreMesh(core_axis_name="c", subcore_axis_name="s"),
    scratch_types=[
        pltpu.VMEM((_BS,), jnp.int32),
        pltpu.VMEM((_EPW, _K), jnp.float32),
        pltpu.VMEM((2, _EPW, _S), jnp.float32),
        pltpu.SemaphoreType.DMA,
    ],
    compiler_params=pltpu.CompilerParams(needs_layout_passes=False),
)(_gather_body)


def _commit_body(z_ref, enc_ref, commit_ref, loss_ref, acc_ref):
    i = pl.program_id(0)

    @pl.when(i == 0)
    def _init():
        acc_ref[0, 0] = 0.0

    diff = z_ref[...] - enc_ref[...]
    acc_ref[0, 0] += jnp.sum(diff * diff)

    @pl.when(i == _B - 1)
    def _fin():
        c = acc_ref[0, 0] / jnp.float32(_NELEMS)
        commit_ref[0, 0] = c
        loss_ref[0, 0] = jnp.float32(_COMMIT) * c


def _tc_commit(inputs, output):
    return pl.pallas_call(
        _commit_body,
        grid=(_B,),
        in_specs=[
            pl.BlockSpec((1, _H, _S), lambda i: (i, 0, 0)),
            pl.BlockSpec((1, _H, _S), lambda i: (i, 0, 0)),
        ],
        out_specs=[
            pl.BlockSpec(memory_space=pltpu.SMEM),
            pl.BlockSpec(memory_space=pltpu.SMEM),
        ],
        out_shape=[
            jax.ShapeDtypeStruct((1, 1), jnp.float32),
            jax.ShapeDtypeStruct((1, 1), jnp.float32),
        ],
        scratch_shapes=[pltpu.SMEM((1, 1), jnp.float32)],
    )(inputs, output)


def kernel(inputs, embeddings):
    zrow = inputs.reshape(_BS, _ND * _ED)
    idx = _tc_argmin(zrow, embeddings)
    emb_flat = embeddings.reshape(_ND * _ED, _K)
    output = _sc_gather(emb_flat, idx)
    commit, loss = _tc_commit(inputs, output)
    return (output, loss[0, 0], commit[0, 0], jnp.array(0))


# speedup vs baseline: 1.6629x; 1.0074x over previous
import functools

import jax
import jax.numpy as jnp
from jax.experimental import pallas as pl
from jax.experimental.pallas import tpu as pltpu
from jax.experimental.pallas import tpu_sc as plsc

_B, _H, _S = 32, 1024, 576
_ND, _ED, _K = 4, 256, 1024
_COMMIT = 0.25
_BS = _B * _S
_TB = 1024
_NSTEPS = _BS // _TB
_NELEMS = _B * _H * _S
_NW = 32
_EPW = _ED // 8
_L = 16

_PREC_DIST = jax.lax.Precision.DEFAULT


def _argmin_body(zrow_ref, emb_ref, idx_ref, e2_ref):
    i = pl.program_id(0)

    @pl.when(i == 0)
    def _init():
        e2_ref[...] = jnp.sum(emb_ref[...] * emb_ref[...], axis=1)

    for d in range(_ND):
        zd = zrow_ref[:, d * _ED:(d + 1) * _ED]
        emb = emb_ref[d]
        e2 = e2_ref[d]
        z2 = jnp.sum(zd * zd, axis=1)
        g = jax.lax.dot_general(
            zd, emb, (((1,), (0,)), ((), ())),
            preferred_element_type=jnp.float32,
            precision=_PREC_DIST)
        scores = (z2[:, None] + e2[None, :]) - 2.0 * g
        m = jnp.min(scores, axis=1, keepdims=True)
        kiota = jax.lax.broadcasted_iota(
            jnp.int32, (_TB, _K), 1).astype(jnp.float32)
        idx_ref[d] = jnp.min(
            jnp.where(scores == m, kiota, jnp.float32(_K)),
            axis=1).astype(jnp.int32)


def _tc_argmin(zrow, embeddings):
    return pl.pallas_call(
        _argmin_body,
        grid=(_NSTEPS,),
        in_specs=[
            pl.BlockSpec((_TB, _ND * _ED), lambda i: (i, 0)),
            pl.BlockSpec((_ND, _ED, _K), lambda i: (0, 0, 0)),
        ],
        out_specs=pl.BlockSpec((8, _TB), lambda i: (0, i)),
        out_shape=jax.ShapeDtypeStruct((8, _BS), jnp.int32),
        scratch_shapes=[pltpu.VMEM((_ND, _K), jnp.float32)],
    )(zrow, embeddings)


def _gather_body(emb_ref, idx_ref, out_ref, idx_v, tab_v, o_v, sem):
    wid = jax.lax.axis_index("s") * 2 + jax.lax.axis_index("c")
    d = wid // 8
    e0 = (wid % 8) * _EPW
    pltpu.sync_copy(idx_ref.at[d], idx_v)
    pltpu.sync_copy(emb_ref.at[pl.ds(d * _ED + e0, _EPW)], tab_v)

    def j_body(j, carry):
        jv = jnp.full((_L,), j, jnp.int32)
        buf = jax.lax.rem(j, 2)

        def r_body(r, inner):
            for s16 in range(_S // _L):
                t0 = r * _S + s16 * _L
                i16 = idx_v[pl.ds(t0, _L)]
                g = plsc.load_gather(tab_v, [jv, i16])
                o_v[buf, r, pl.ds(s16 * _L, _L)] = g
            return inner

        plsc.parallel_loop(0, _EPW, unroll=8, carry=jnp.int32(0))(r_body)

        @pl.when(j > 0)
        def _drain():
            pltpu.make_async_copy(
                o_v.at[1 - buf],
                out_ref.at[wid, pl.ds((j - 1) * _EPW, _EPW)], sem).wait()

        pltpu.make_async_copy(
            o_v.at[buf],
            out_ref.at[wid, pl.ds(j * _EPW, _EPW)], sem).start()
        return carry

    jax.lax.fori_loop(0, _EPW, j_body, jnp.int32(0))
    pltpu.make_async_copy(
        o_v.at[1], out_ref.at[wid, pl.ds((_EPW - 1) * _EPW, _EPW)],
        sem).wait()


_sc_gather = functools.partial(
    pl.kernel,
    out_type=jax.ShapeDtypeStruct((_B, _H, _S), jnp.float32),
    mesh=plsc.VectorSubcoreMesh(core_axis_name="c", subcore_axis_name="s"),
    scratch_types=[
        pltpu.VMEM((_BS,), jnp.int32),
        pltpu.VMEM((_EPW, _K), jnp.float32),
        pltpu.VMEM((2, _EPW, _S), jnp.float32),
        pltpu.SemaphoreType.DMA,
    ],
    compiler_params=pltpu.CompilerParams(needs_layout_passes=False),
)(_gather_body)


def _commit_body(z_ref, enc_ref, commit_ref, loss_ref, acc_ref):
    i = pl.program_id(0)

    @pl.when(i == 0)
    def _init():
        acc_ref[0, 0] = 0.0

    diff = z_ref[...] - enc_ref[...]
    acc_ref[0, 0] += jnp.sum(diff * diff)

    @pl.when(i == _B - 1)
    def _fin():
        c = acc_ref[0, 0] / jnp.float32(_NELEMS)
        commit_ref[0, 0] = c
        loss_ref[0, 0] = jnp.float32(_COMMIT) * c


def _tc_commit(inputs, output):
    return pl.pallas_call(
        _commit_body,
        grid=(_B,),
        in_specs=[
            pl.BlockSpec((1, _H, _S), lambda i: (i, 0, 0)),
            pl.BlockSpec((1, _H, _S), lambda i: (i, 0, 0)),
        ],
        out_specs=[
            pl.BlockSpec(memory_space=pltpu.SMEM),
            pl.BlockSpec(memory_space=pltpu.SMEM),
        ],
        out_shape=[
            jax.ShapeDtypeStruct((1, 1), jnp.float32),
            jax.ShapeDtypeStruct((1, 1), jnp.float32),
        ],
        scratch_shapes=[pltpu.SMEM((1, 1), jnp.float32)],
    )(inputs, output)


def kernel(inputs, embeddings):
    zrow = inputs.reshape(_BS, _ND * _ED)
    idx = _tc_argmin(zrow, embeddings)
    emb_flat = embeddings.reshape(_ND * _ED, _K)
    output = _sc_gather(emb_flat, idx)
    commit, loss = _tc_commit(inputs, output)
    return (output, loss[0, 0], commit[0, 0], jnp.array(0))
